# baseline (speedup 1.0000x reference)
import functools

import jax
import jax.numpy as jnp
from jax import lax
from jax.experimental import pallas as pl
from jax.experimental.pallas import tpu as pltpu
from jax.experimental.pallas import tpu_sc as plsc

_IN = 2048
_OUT = 2048
_NC = 1024
_TOPK = 8
_TOKENS = 8192
_TILE = 256


def _fused_body(x_ref, rw_ref, u_ref, svh_ref, pw_ref, o_ref):
    xb = x_ref[...]

    logits = jax.lax.dot_general(
        xb, rw_ref[...], (((1,), (1,)), ((), ())),
        preferred_element_type=jnp.float32,
    )
    a = jnp.abs(logits)

    cur = a
    thr = jnp.zeros((a.shape[0], 1), jnp.float32)
    for _ in range(_TOPK):
        thr = jnp.max(cur, axis=1, keepdims=True)
        cur = jnp.where(cur >= thr, -jnp.inf, cur)

    dots = jax.lax.dot_general(
        xb, svh_ref[...], (((1,), (1,)), ((), ())),
        preferred_element_type=jnp.float32,
    )
    masked = jnp.where(a >= thr, dots, 0.0)

    base = jax.lax.dot_general(
        xb, pw_ref[...], (((1,), (1,)), ((), ())),
        preferred_element_type=jnp.float32,
    )
    expert = jax.lax.dot_general(
        masked, u_ref[...], (((1,), (1,)), ((), ())),
        preferred_element_type=jnp.float32,
    )
    o_ref[...] = base + expert


def _fused_tc(x, router_w, u, svh, pretrained_w):
    grid = (_TOKENS // _TILE,)
    return pl.pallas_call(
        _fused_body,
        grid=grid,
        in_specs=[
            pl.BlockSpec((_TILE, _IN), lambda i: (i, 0)),
            pl.BlockSpec((_NC, _IN), lambda i: (0, 0)),
            pl.BlockSpec((_OUT, _NC), lambda i: (0, 0)),
            pl.BlockSpec((_NC, _IN), lambda i: (0, 0)),
            pl.BlockSpec((_OUT, _IN), lambda i: (0, 0)),
        ],
        out_specs=pl.BlockSpec((_TILE, _OUT), lambda i: (i, 0)),
        out_shape=jax.ShapeDtypeStruct((_TOKENS, _OUT), jnp.float32),
        compiler_params=pltpu.CompilerParams(
            dimension_semantics=("arbitrary",),
            vmem_limit_bytes=100 * 1024 * 1024,
        ),
    )(x, router_w, u, svh, pretrained_w)


def _stage_a_body(x_ref, rw_ref, svh_ref, aT_ref, dT_ref):
    xb = x_ref[...]
    logitsT = jax.lax.dot_general(
        rw_ref[...], xb, (((1,), (1,)), ((), ())),
        preferred_element_type=jnp.float32,
    )
    aT_ref[...] = jnp.abs(logitsT)
    dT_ref[...] = jax.lax.dot_general(
        svh_ref[...], xb, (((1,), (1,)), ((), ())),
        preferred_element_type=jnp.float32,
    )


def _stage_a(x, router_w, svh):
    grid = (_TOKENS // _TILE,)
    return pl.pallas_call(
        _stage_a_body,
        grid=grid,
        in_specs=[
            pl.BlockSpec((_TILE, _IN), lambda i: (i, 0)),
            pl.BlockSpec((_NC, _IN), lambda i: (0, 0)),
            pl.BlockSpec((_NC, _IN), lambda i: (0, 0)),
        ],
        out_specs=[
            pl.BlockSpec((_NC, _TILE), lambda i: (0, i)),
            pl.BlockSpec((_NC, _TILE), lambda i: (0, i)),
        ],
        out_shape=[
            jax.ShapeDtypeStruct((_NC, _TOKENS), jnp.float32),
            jax.ShapeDtypeStruct((_NC, _TOKENS), jnp.float32),
        ],
        compiler_params=pltpu.CompilerParams(
            dimension_semantics=("arbitrary",),
            vmem_limit_bytes=100 * 1024 * 1024,
        ),
    )(x, router_w, svh)


_SC_LANES = 16
_SC_WORKERS = 32
_ROWS_PER_W = _TOKENS // _SC_WORKERS
_CHUNK = 128
_N_CG = _ROWS_PER_W // _CHUNK
_N_RC = _NC // _CHUNK
_N_LS = _CHUNK // _SC_LANES


def _sc_thr_kernel(aT_hbm, thr_hbm, buf_v, state_v, thr_v):
    wid = lax.axis_index("s") * 2 + lax.axis_index("c")
    neg_inf = jnp.full((_SC_LANES,), -jnp.inf, dtype=jnp.float32)

    for cg in range(_N_CG):
        col0 = pl.multiple_of((wid * _N_CG + cg) * _CHUNK, _CHUNK)
        for ls in range(_N_LS):
            for i in range(_TOPK):
                state_v[ls, i, :] = neg_inf
        for rc in range(_N_RC):
            pltpu.sync_copy(
                aT_hbm.at[pl.ds(rc * _CHUNK, _CHUNK), pl.ds(col0, _CHUNK)],
                buf_v,
            )
            for ls in range(_N_LS):
                def body(j, t, _ls=ls):
                    v = buf_v[j, pl.ds(_ls * _SC_LANES, _SC_LANES)]
                    new_t = []
                    for i in range(_TOPK):
                        hi = jnp.maximum(t[i], v)
                        v = jnp.minimum(t[i], v)
                        new_t.append(hi)
                    return tuple(new_t)

                t0 = tuple(state_v[ls, i, :] for i in range(_TOPK))
                t = lax.fori_loop(0, _CHUNK, body, t0)
                for i in range(_TOPK):
                    state_v[ls, i, :] = t[i]
        for ls in range(_N_LS):
            thr_v[pl.ds((cg * _N_LS + ls) * _SC_LANES, _SC_LANES)] = (
                state_v[ls, _TOPK - 1, :]
            )

    pltpu.sync_copy(thr_v, thr_hbm.at[pl.ds(wid * _ROWS_PER_W, _ROWS_PER_W)])


def _sc_threshold(aT):
    mesh = plsc.VectorSubcoreMesh(core_axis_name="c", subcore_axis_name="s")
    fn = functools.partial(
        pl.kernel,
        out_type=jax.ShapeDtypeStruct((_TOKENS,), jnp.float32),
        mesh=mesh,
        scratch_types=[
            pltpu.VMEM((_CHUNK, _CHUNK), jnp.float32),
            pltpu.VMEM((_N_LS, _TOPK, _SC_LANES), jnp.float32),
            pltpu.VMEM((_ROWS_PER_W,), jnp.float32),
        ],
    )(_sc_thr_kernel)
    return fn(aT)


def _stage_b_body(x_ref, aT_ref, dT_ref, thr_ref, u_ref, pw_ref, o_ref):
    xb = x_ref[...]
    maskedT = jnp.where(aT_ref[...] >= thr_ref[...], dT_ref[...], 0.0)
    base = jax.lax.dot_general(
        xb, pw_ref[...], (((1,), (1,)), ((), ())),
        preferred_element_type=jnp.float32,
    )
    expert = jax.lax.dot_general(
        maskedT, u_ref[...], (((0,), (1,)), ((), ())),
        preferred_element_type=jnp.float32,
    )
    o_ref[...] = base + expert


def _stage_b(x, aT, dT, thr, u, pretrained_w):
    grid = (_TOKENS // _TILE,)
    return pl.pallas_call(
        _stage_b_body,
        grid=grid,
        in_specs=[
            pl.BlockSpec((_TILE, _IN), lambda i: (i, 0)),
            pl.BlockSpec((_NC, _TILE), lambda i: (0, i)),
            pl.BlockSpec((_NC, _TILE), lambda i: (0, i)),
            pl.BlockSpec((1, _TILE), lambda i: (0, i)),
            pl.BlockSpec((_OUT, _NC), lambda i: (0, 0)),
            pl.BlockSpec((_OUT, _IN), lambda i: (0, 0)),
        ],
        out_specs=pl.BlockSpec((_TILE, _OUT), lambda i: (i, 0)),
        out_shape=jax.ShapeDtypeStruct((_TOKENS, _OUT), jnp.float32),
        compiler_params=pltpu.CompilerParams(
            dimension_semantics=("arbitrary",),
            vmem_limit_bytes=100 * 1024 * 1024,
        ),
    )(x, aT, dT, thr, u, pretrained_w)


@jax.jit
def kernel(x, router_w, u, svh, pretrained_w):
    aT, dT = _stage_a(x, router_w, svh)
    thr = _sc_threshold(aT)
    return _stage_b(x, aT, dT, thr.reshape(1, _TOKENS), u, pretrained_w)


# speedup vs baseline: 1.1640x; 1.1640x over previous
import functools

import jax
import jax.numpy as jnp
from jax import lax
from jax.experimental import pallas as pl
from jax.experimental.pallas import tpu as pltpu
from jax.experimental.pallas import tpu_sc as plsc

_IN = 2048
_OUT = 2048
_NC = 1024
_TOPK = 8
_TOKENS = 8192
_TILE = 256


def _fused_body(x_ref, rw_ref, u_ref, svh_ref, pw_ref, o_ref):
    xb = x_ref[...]

    logits = jax.lax.dot_general(
        xb, rw_ref[...], (((1,), (1,)), ((), ())),
        preferred_element_type=jnp.float32,
    )
    a = jnp.abs(logits)

    cur = a
    thr = jnp.zeros((a.shape[0], 1), jnp.float32)
    for _ in range(_TOPK):
        thr = jnp.max(cur, axis=1, keepdims=True)
        cur = jnp.where(cur >= thr, -jnp.inf, cur)

    dots = jax.lax.dot_general(
        xb, svh_ref[...], (((1,), (1,)), ((), ())),
        preferred_element_type=jnp.float32,
    )
    masked = jnp.where(a >= thr, dots, 0.0)

    base = jax.lax.dot_general(
        xb, pw_ref[...], (((1,), (1,)), ((), ())),
        preferred_element_type=jnp.float32,
    )
    expert = jax.lax.dot_general(
        masked, u_ref[...], (((1,), (1,)), ((), ())),
        preferred_element_type=jnp.float32,
    )
    o_ref[...] = base + expert


def _fused_tc(x, router_w, u, svh, pretrained_w):
    grid = (_TOKENS // _TILE,)
    return pl.pallas_call(
        _fused_body,
        grid=grid,
        in_specs=[
            pl.BlockSpec((_TILE, _IN), lambda i: (i, 0)),
            pl.BlockSpec((_NC, _IN), lambda i: (0, 0)),
            pl.BlockSpec((_OUT, _NC), lambda i: (0, 0)),
            pl.BlockSpec((_NC, _IN), lambda i: (0, 0)),
            pl.BlockSpec((_OUT, _IN), lambda i: (0, 0)),
        ],
        out_specs=pl.BlockSpec((_TILE, _OUT), lambda i: (i, 0)),
        out_shape=jax.ShapeDtypeStruct((_TOKENS, _OUT), jnp.float32),
        compiler_params=pltpu.CompilerParams(
            dimension_semantics=("arbitrary",),
            vmem_limit_bytes=100 * 1024 * 1024,
        ),
    )(x, router_w, u, svh, pretrained_w)


def _stage_a_body(x_ref, rw_ref, svh_ref, aT_ref, dT_ref):
    xb = x_ref[...]
    logitsT = jax.lax.dot_general(
        rw_ref[...], xb, (((1,), (1,)), ((), ())),
        preferred_element_type=jnp.float32,
    )
    aT_ref[...] = jnp.abs(logitsT)
    dT_ref[...] = jax.lax.dot_general(
        svh_ref[...], xb, (((1,), (1,)), ((), ())),
        preferred_element_type=jnp.float32,
    )


def _stage_a(x, router_w, svh):
    grid = (_TOKENS // _TILE,)
    return pl.pallas_call(
        _stage_a_body,
        grid=grid,
        in_specs=[
            pl.BlockSpec((_TILE, _IN), lambda i: (i, 0)),
            pl.BlockSpec((_NC, _IN), lambda i: (0, 0)),
            pl.BlockSpec((_NC, _IN), lambda i: (0, 0)),
        ],
        out_specs=[
            pl.BlockSpec((_NC, _TILE), lambda i: (0, i)),
            pl.BlockSpec((_NC, _TILE), lambda i: (0, i)),
        ],
        out_shape=[
            jax.ShapeDtypeStruct((_NC, _TOKENS), jnp.float32),
            jax.ShapeDtypeStruct((_NC, _TOKENS), jnp.float32),
        ],
        compiler_params=pltpu.CompilerParams(
            dimension_semantics=("arbitrary",),
            vmem_limit_bytes=100 * 1024 * 1024,
        ),
    )(x, router_w, svh)


_SC_LANES = 16
_SC_WORKERS = 32
_ROWS_PER_W = _TOKENS // _SC_WORKERS
_CHUNK = 128
_N_CG = _ROWS_PER_W // _CHUNK
_N_RC = _NC // _CHUNK
_N_LS = _CHUNK // _SC_LANES


def _sc_thr_kernel(aT_hbm, thr_hbm, buf_v, state_v, thr_v):
    wid = lax.axis_index("s") * 2 + lax.axis_index("c")
    neg_inf = jnp.full((_SC_LANES,), -jnp.inf, dtype=jnp.float32)

    for cg in range(_N_CG):
        col0 = pl.multiple_of((wid * _N_CG + cg) * _CHUNK, _CHUNK)
        for ls in range(_N_LS):
            for i in range(_TOPK):
                state_v[ls, i, :] = neg_inf
        for rc in range(_N_RC):
            pltpu.sync_copy(
                aT_hbm.at[pl.ds(rc * _CHUNK, _CHUNK), pl.ds(col0, _CHUNK)],
                buf_v,
            )
            for ls in range(_N_LS):
                def body(j, t, _ls=ls):
                    v = buf_v[j, pl.ds(_ls * _SC_LANES, _SC_LANES)]
                    new_t = []
                    for i in range(_TOPK):
                        hi = jnp.maximum(t[i], v)
                        v = jnp.minimum(t[i], v)
                        new_t.append(hi)
                    return tuple(new_t)

                t0 = tuple(state_v[ls, i, :] for i in range(_TOPK))
                t = lax.fori_loop(0, _CHUNK, body, t0)
                for i in range(_TOPK):
                    state_v[ls, i, :] = t[i]
        for ls in range(_N_LS):
            thr_v[pl.ds((cg * _N_LS + ls) * _SC_LANES, _SC_LANES)] = (
                state_v[ls, _TOPK - 1, :]
            )

    pltpu.sync_copy(thr_v, thr_hbm.at[pl.ds(wid * _ROWS_PER_W, _ROWS_PER_W)])


def _sc_threshold(aT):
    mesh = plsc.VectorSubcoreMesh(core_axis_name="c", subcore_axis_name="s")
    fn = functools.partial(
        pl.kernel,
        out_type=jax.ShapeDtypeStruct((_TOKENS,), jnp.float32),
        mesh=mesh,
        scratch_types=[
            pltpu.VMEM((_CHUNK, _CHUNK), jnp.float32),
            pltpu.VMEM((_N_LS, _TOPK, _SC_LANES), jnp.float32),
            pltpu.VMEM((_ROWS_PER_W,), jnp.float32),
        ],
    )(_sc_thr_kernel)
    return fn(aT)


def _base_body(x_ref, pw_ref, b_ref):
    b_ref[...] = jax.lax.dot_general(
        x_ref[...], pw_ref[...], (((1,), (1,)), ((), ())),
        preferred_element_type=jnp.float32,
    )


def _stage_base(x, pretrained_w):
    grid = (_TOKENS // _TILE,)
    return pl.pallas_call(
        _base_body,
        grid=grid,
        in_specs=[
            pl.BlockSpec((_TILE, _IN), lambda i: (i, 0)),
            pl.BlockSpec((_OUT, _IN), lambda i: (0, 0)),
        ],
        out_specs=pl.BlockSpec((_TILE, _OUT), lambda i: (i, 0)),
        out_shape=jax.ShapeDtypeStruct((_TOKENS, _OUT), jnp.float32),
        compiler_params=pltpu.CompilerParams(
            dimension_semantics=("arbitrary",),
            vmem_limit_bytes=100 * 1024 * 1024,
        ),
    )(x, pretrained_w)


def _stage_b_body(base_ref, aT_ref, dT_ref, thr_ref, u_ref, o_ref):
    maskedT = jnp.where(aT_ref[...] >= thr_ref[...], dT_ref[...], 0.0)
    expert = jax.lax.dot_general(
        maskedT, u_ref[...], (((0,), (1,)), ((), ())),
        preferred_element_type=jnp.float32,
    )
    o_ref[...] = base_ref[...] + expert


def _stage_b(base, aT, dT, thr, u):
    grid = (_TOKENS // _TILE,)
    return pl.pallas_call(
        _stage_b_body,
        grid=grid,
        in_specs=[
            pl.BlockSpec((_TILE, _OUT), lambda i: (i, 0)),
            pl.BlockSpec((_NC, _TILE), lambda i: (0, i)),
            pl.BlockSpec((_NC, _TILE), lambda i: (0, i)),
            pl.BlockSpec((1, _TILE), lambda i: (0, i)),
            pl.BlockSpec((_OUT, _NC), lambda i: (0, 0)),
        ],
        out_specs=pl.BlockSpec((_TILE, _OUT), lambda i: (i, 0)),
        out_shape=jax.ShapeDtypeStruct((_TOKENS, _OUT), jnp.float32),
        compiler_params=pltpu.CompilerParams(
            dimension_semantics=("arbitrary",),
            vmem_limit_bytes=100 * 1024 * 1024,
        ),
    )(base, aT, dT, thr, u)


@jax.jit
def kernel(x, router_w, u, svh, pretrained_w):
    aT, dT = _stage_a(x, router_w, svh)
    thr = _sc_threshold(aT)
    base = _stage_base(x, pretrained_w)
    return _stage_b(base, aT, dT, thr.reshape(1, _TOKENS), u)


# speedup vs baseline: 1.1800x; 1.0137x over previous
import functools

import jax
import jax.numpy as jnp
from jax import lax
from jax.experimental import pallas as pl
from jax.experimental.pallas import tpu as pltpu
from jax.experimental.pallas import tpu_sc as plsc

_IN = 2048
_OUT = 2048
_NC = 1024
_TOPK = 8
_TOKENS = 8192
_TILE = 256


def _fused_body(x_ref, rw_ref, u_ref, svh_ref, pw_ref, o_ref):
    xb = x_ref[...]

    logits = jax.lax.dot_general(
        xb, rw_ref[...], (((1,), (1,)), ((), ())),
        preferred_element_type=jnp.float32,
    )
    a = jnp.abs(logits)

    cur = a
    thr = jnp.zeros((a.shape[0], 1), jnp.float32)
    for _ in range(_TOPK):
        thr = jnp.max(cur, axis=1, keepdims=True)
        cur = jnp.where(cur >= thr, -jnp.inf, cur)

    dots = jax.lax.dot_general(
        xb, svh_ref[...], (((1,), (1,)), ((), ())),
        preferred_element_type=jnp.float32,
    )
    masked = jnp.where(a >= thr, dots, 0.0)

    base = jax.lax.dot_general(
        xb, pw_ref[...], (((1,), (1,)), ((), ())),
        preferred_element_type=jnp.float32,
    )
    expert = jax.lax.dot_general(
        masked, u_ref[...], (((1,), (1,)), ((), ())),
        preferred_element_type=jnp.float32,
    )
    o_ref[...] = base + expert


def _fused_tc(x, router_w, u, svh, pretrained_w):
    grid = (_TOKENS // _TILE,)
    return pl.pallas_call(
        _fused_body,
        grid=grid,
        in_specs=[
            pl.BlockSpec((_TILE, _IN), lambda i: (i, 0)),
            pl.BlockSpec((_NC, _IN), lambda i: (0, 0)),
            pl.BlockSpec((_OUT, _NC), lambda i: (0, 0)),
            pl.BlockSpec((_NC, _IN), lambda i: (0, 0)),
            pl.BlockSpec((_OUT, _IN), lambda i: (0, 0)),
        ],
        out_specs=pl.BlockSpec((_TILE, _OUT), lambda i: (i, 0)),
        out_shape=jax.ShapeDtypeStruct((_TOKENS, _OUT), jnp.float32),
        compiler_params=pltpu.CompilerParams(
            dimension_semantics=("arbitrary",),
            vmem_limit_bytes=100 * 1024 * 1024,
        ),
    )(x, router_w, u, svh, pretrained_w)


def _stage_a_body(x_ref, rw_ref, svh_ref, aT_ref, dT_ref):
    xb = x_ref[...]
    logitsT = jax.lax.dot_general(
        rw_ref[...], xb, (((1,), (1,)), ((), ())),
        preferred_element_type=jnp.float32,
    )
    aT_ref[...] = jnp.abs(logitsT)
    dT_ref[...] = jax.lax.dot_general(
        svh_ref[...], xb, (((1,), (1,)), ((), ())),
        preferred_element_type=jnp.float32,
    ).astype(jnp.bfloat16)


def _stage_a(x, router_w, svh):
    grid = (_TOKENS // _TILE,)
    return pl.pallas_call(
        _stage_a_body,
        grid=grid,
        in_specs=[
            pl.BlockSpec((_TILE, _IN), lambda i: (i, 0)),
            pl.BlockSpec((_NC, _IN), lambda i: (0, 0)),
            pl.BlockSpec((_NC, _IN), lambda i: (0, 0)),
        ],
        out_specs=[
            pl.BlockSpec((_NC, _TILE), lambda i: (0, i)),
            pl.BlockSpec((_NC, _TILE), lambda i: (0, i)),
        ],
        out_shape=[
            jax.ShapeDtypeStruct((_NC, _TOKENS), jnp.float32),
            jax.ShapeDtypeStruct((_NC, _TOKENS), jnp.bfloat16),
        ],
        compiler_params=pltpu.CompilerParams(
            dimension_semantics=("arbitrary",),
            vmem_limit_bytes=100 * 1024 * 1024,
        ),
    )(x, router_w, svh)


_SC_LANES = 16
_SC_WORKERS = 32
_ROWS_PER_W = _TOKENS // _SC_WORKERS
_CHUNK = 128
_N_CG = _ROWS_PER_W // _CHUNK
_N_RC = _NC // _CHUNK
_N_LS = _CHUNK // _SC_LANES


def _sc_thr_kernel(aT_hbm, thr_hbm, buf_v, state_v, thr_v):
    wid = lax.axis_index("s") * 2 + lax.axis_index("c")
    neg_inf = jnp.full((_SC_LANES,), -jnp.inf, dtype=jnp.float32)

    for cg in range(_N_CG):
        col0 = pl.multiple_of((wid * _N_CG + cg) * _CHUNK, _CHUNK)
        for ls in range(_N_LS):
            for i in range(_TOPK):
                state_v[ls, i, :] = neg_inf
        for rc in range(_N_RC):
            pltpu.sync_copy(
                aT_hbm.at[pl.ds(rc * _CHUNK, _CHUNK), pl.ds(col0, _CHUNK)],
                buf_v,
            )
            for ls in range(_N_LS):
                def body(j, t, _ls=ls):
                    v = buf_v[j, pl.ds(_ls * _SC_LANES, _SC_LANES)]
                    new_t = []
                    for i in range(_TOPK):
                        hi = jnp.maximum(t[i], v)
                        v = jnp.minimum(t[i], v)
                        new_t.append(hi)
                    return tuple(new_t)

                t0 = tuple(state_v[ls, i, :] for i in range(_TOPK))
                t = lax.fori_loop(0, _CHUNK, body, t0)
                for i in range(_TOPK):
                    state_v[ls, i, :] = t[i]
        for ls in range(_N_LS):
            thr_v[pl.ds((cg * _N_LS + ls) * _SC_LANES, _SC_LANES)] = (
                state_v[ls, _TOPK - 1, :]
            )

    pltpu.sync_copy(thr_v, thr_hbm.at[pl.ds(wid * _ROWS_PER_W, _ROWS_PER_W)])


def _sc_threshold(aT):
    mesh = plsc.VectorSubcoreMesh(core_axis_name="c", subcore_axis_name="s")
    fn = functools.partial(
        pl.kernel,
        out_type=jax.ShapeDtypeStruct((_TOKENS,), jnp.float32),
        mesh=mesh,
        scratch_types=[
            pltpu.VMEM((_CHUNK, _CHUNK), jnp.float32),
            pltpu.VMEM((_N_LS, _TOPK, _SC_LANES), jnp.float32),
            pltpu.VMEM((_ROWS_PER_W,), jnp.float32),
        ],
    )(_sc_thr_kernel)
    return fn(aT)


def _base_body(x_ref, pw_ref, b_ref):
    b_ref[...] = jax.lax.dot_general(
        x_ref[...], pw_ref[...], (((1,), (1,)), ((), ())),
        preferred_element_type=jnp.float32,
    ).astype(jnp.bfloat16)


def _stage_base(x, pretrained_w):
    grid = (_TOKENS // _TILE,)
    return pl.pallas_call(
        _base_body,
        grid=grid,
        in_specs=[
            pl.BlockSpec((_TILE, _IN), lambda i: (i, 0)),
            pl.BlockSpec((_OUT, _IN), lambda i: (0, 0)),
        ],
        out_specs=pl.BlockSpec((_TILE, _OUT), lambda i: (i, 0)),
        out_shape=jax.ShapeDtypeStruct((_TOKENS, _OUT), jnp.bfloat16),
        compiler_params=pltpu.CompilerParams(
            dimension_semantics=("arbitrary",),
            vmem_limit_bytes=100 * 1024 * 1024,
        ),
    )(x, pretrained_w)


def _stage_b_body(base_ref, aT_ref, dT_ref, thr_ref, u_ref, o_ref):
    zero = jnp.zeros((), jnp.bfloat16)
    maskedT = jnp.where(aT_ref[...] >= thr_ref[...], dT_ref[...], zero)
    expert = jax.lax.dot_general(
        maskedT, u_ref[...], (((0,), (1,)), ((), ())),
        preferred_element_type=jnp.float32,
    )
    o_ref[...] = base_ref[...].astype(jnp.float32) + expert


def _stage_b(base, aT, dT, thr, u):
    grid = (_TOKENS // _TILE,)
    return pl.pallas_call(
        _stage_b_body,
        grid=grid,
        in_specs=[
            pl.BlockSpec((_TILE, _OUT), lambda i: (i, 0)),
            pl.BlockSpec((_NC, _TILE), lambda i: (0, i)),
            pl.BlockSpec((_NC, _TILE), lambda i: (0, i)),
            pl.BlockSpec((1, _TILE), lambda i: (0, i)),
            pl.BlockSpec((_OUT, _NC), lambda i: (0, 0)),
        ],
        out_specs=pl.BlockSpec((_TILE, _OUT), lambda i: (i, 0)),
        out_shape=jax.ShapeDtypeStruct((_TOKENS, _OUT), jnp.float32),
        compiler_params=pltpu.CompilerParams(
            dimension_semantics=("arbitrary",),
            vmem_limit_bytes=100 * 1024 * 1024,
        ),
    )(base, aT, dT, thr, u)


@jax.jit
def kernel(x, router_w, u, svh, pretrained_w):
    aT, dT = _stage_a(x, router_w, svh)
    thr = _sc_threshold(aT)
    base = _stage_base(x, pretrained_w)
    return _stage_b(base, aT, dT, thr.reshape(1, _TOKENS),
                    u.astype(jnp.bfloat16))


# speedup vs baseline: 1.1819x; 1.0016x over previous
import functools

import jax
import jax.numpy as jnp
from jax import lax
from jax.experimental import pallas as pl
from jax.experimental.pallas import tpu as pltpu
from jax.experimental.pallas import tpu_sc as plsc

_IN = 2048
_OUT = 2048
_NC = 1024
_TOPK = 8
_TOKENS = 8192
_TILE = 256


def _fused_body(x_ref, rw_ref, u_ref, svh_ref, pw_ref, o_ref):
    xb = x_ref[...]

    logits = jax.lax.dot_general(
        xb, rw_ref[...], (((1,), (1,)), ((), ())),
        preferred_element_type=jnp.float32,
    )
    a = jnp.abs(logits)

    cur = a
    thr = jnp.zeros((a.shape[0], 1), jnp.float32)
    for _ in range(_TOPK):
        thr = jnp.max(cur, axis=1, keepdims=True)
        cur = jnp.where(cur >= thr, -jnp.inf, cur)

    dots = jax.lax.dot_general(
        xb, svh_ref[...], (((1,), (1,)), ((), ())),
        preferred_element_type=jnp.float32,
    )
    masked = jnp.where(a >= thr, dots, 0.0)

    base = jax.lax.dot_general(
        xb, pw_ref[...], (((1,), (1,)), ((), ())),
        preferred_element_type=jnp.float32,
    )
    expert = jax.lax.dot_general(
        masked, u_ref[...], (((1,), (1,)), ((), ())),
        preferred_element_type=jnp.float32,
    )
    o_ref[...] = base + expert


def _fused_tc(x, router_w, u, svh, pretrained_w):
    grid = (_TOKENS // _TILE,)
    return pl.pallas_call(
        _fused_body,
        grid=grid,
        in_specs=[
            pl.BlockSpec((_TILE, _IN), lambda i: (i, 0)),
            pl.BlockSpec((_NC, _IN), lambda i: (0, 0)),
            pl.BlockSpec((_OUT, _NC), lambda i: (0, 0)),
            pl.BlockSpec((_NC, _IN), lambda i: (0, 0)),
            pl.BlockSpec((_OUT, _IN), lambda i: (0, 0)),
        ],
        out_specs=pl.BlockSpec((_TILE, _OUT), lambda i: (i, 0)),
        out_shape=jax.ShapeDtypeStruct((_TOKENS, _OUT), jnp.float32),
        compiler_params=pltpu.CompilerParams(
            dimension_semantics=("arbitrary",),
            vmem_limit_bytes=100 * 1024 * 1024,
        ),
    )(x, router_w, u, svh, pretrained_w)


def _stage_a_body(x_ref, rw_ref, svh_ref, aT_ref, dT_ref):
    xb = x_ref[...]
    logitsT = jax.lax.dot_general(
        rw_ref[...], xb, (((1,), (1,)), ((), ())),
        preferred_element_type=jnp.float32,
    )
    aT_ref[...] = jnp.abs(logitsT)
    dT_ref[...] = jax.lax.dot_general(
        svh_ref[...], xb, (((1,), (1,)), ((), ())),
        preferred_element_type=jnp.float32,
    ).astype(jnp.bfloat16)


def _stage_a(x, router_w, svh):
    grid = (_TOKENS // _TILE,)
    return pl.pallas_call(
        _stage_a_body,
        grid=grid,
        in_specs=[
            pl.BlockSpec((_TILE, _IN), lambda i: (i, 0)),
            pl.BlockSpec((_NC, _IN), lambda i: (0, 0)),
            pl.BlockSpec((_NC, _IN), lambda i: (0, 0)),
        ],
        out_specs=[
            pl.BlockSpec((_NC, _TILE), lambda i: (0, i)),
            pl.BlockSpec((_NC, _TILE), lambda i: (0, i)),
        ],
        out_shape=[
            jax.ShapeDtypeStruct((_NC, _TOKENS), jnp.float32),
            jax.ShapeDtypeStruct((_NC, _TOKENS), jnp.bfloat16),
        ],
        compiler_params=pltpu.CompilerParams(
            dimension_semantics=("arbitrary",),
            vmem_limit_bytes=100 * 1024 * 1024,
        ),
    )(x, router_w, svh)


_SC_LANES = 16
_SC_WORKERS = 32
_ROWS_PER_W = _TOKENS // _SC_WORKERS
_CHUNK = 128
_N_CG = _ROWS_PER_W // _CHUNK
_N_RC = _NC // _CHUNK
_N_LS = _CHUNK // _SC_LANES


def _sc_thr_kernel(aT_hbm, thr_hbm, buf0_v, buf1_v, state_v, thr_v,
                   sem0, sem1):
    wid = lax.axis_index("s") * 2 + lax.axis_index("c")
    neg_inf = jnp.full((_SC_LANES,), -jnp.inf, dtype=jnp.float32)
    bufs = (buf0_v, buf1_v)
    sems = (sem0, sem1)

    def start(cg, rc):
        col0 = pl.multiple_of((wid * _N_CG + cg) * _CHUNK, _CHUNK)
        return pltpu.async_copy(
            aT_hbm.at[pl.ds(rc * _CHUNK, _CHUNK), pl.ds(col0, _CHUNK)],
            bufs[rc % 2], sems[rc % 2],
        )

    for cg in range(_N_CG):
        for ls in range(_N_LS):
            for i in range(_TOPK):
                state_v[ls, i, :] = neg_inf
        pending = start(cg, 0)
        for rc in range(_N_RC):
            pending.wait()
            if rc + 1 < _N_RC:
                pending = start(cg, rc + 1)
            buf_v = bufs[rc % 2]
            for ls in range(_N_LS):
                def body(j, t, _b=buf_v, _ls=ls):
                    v = _b[j, pl.ds(_ls * _SC_LANES, _SC_LANES)]
                    new_t = []
                    for i in range(_TOPK):
                        hi = jnp.maximum(t[i], v)
                        v = jnp.minimum(t[i], v)
                        new_t.append(hi)
                    return tuple(new_t)

                t0 = tuple(state_v[ls, i, :] for i in range(_TOPK))
                t = lax.fori_loop(0, _CHUNK, body, t0)
                for i in range(_TOPK):
                    state_v[ls, i, :] = t[i]
        for ls in range(_N_LS):
            thr_v[pl.ds((cg * _N_LS + ls) * _SC_LANES, _SC_LANES)] = (
                state_v[ls, _TOPK - 1, :]
            )

    pltpu.sync_copy(thr_v, thr_hbm.at[pl.ds(wid * _ROWS_PER_W, _ROWS_PER_W)])


def _sc_threshold(aT):
    mesh = plsc.VectorSubcoreMesh(core_axis_name="c", subcore_axis_name="s")
    fn = functools.partial(
        pl.kernel,
        out_type=jax.ShapeDtypeStruct((_TOKENS,), jnp.float32),
        mesh=mesh,
        scratch_types=[
            pltpu.VMEM((_CHUNK, _CHUNK), jnp.float32),
            pltpu.VMEM((_CHUNK, _CHUNK), jnp.float32),
            pltpu.VMEM((_N_LS, _TOPK, _SC_LANES), jnp.float32),
            pltpu.VMEM((_ROWS_PER_W,), jnp.float32),
            pltpu.SemaphoreType.DMA,
            pltpu.SemaphoreType.DMA,
        ],
    )(_sc_thr_kernel)
    return fn(aT)


def _base_body(x_ref, pw_ref, b_ref):
    b_ref[...] = jax.lax.dot_general(
        x_ref[...], pw_ref[...], (((1,), (1,)), ((), ())),
        preferred_element_type=jnp.float32,
    ).astype(jnp.bfloat16)


def _stage_base(x, pretrained_w):
    grid = (_TOKENS // _TILE,)
    return pl.pallas_call(
        _base_body,
        grid=grid,
        in_specs=[
            pl.BlockSpec((_TILE, _IN), lambda i: (i, 0)),
            pl.BlockSpec((_OUT, _IN), lambda i: (0, 0)),
        ],
        out_specs=pl.BlockSpec((_TILE, _OUT), lambda i: (i, 0)),
        out_shape=jax.ShapeDtypeStruct((_TOKENS, _OUT), jnp.bfloat16),
        compiler_params=pltpu.CompilerParams(
            dimension_semantics=("arbitrary",),
            vmem_limit_bytes=100 * 1024 * 1024,
        ),
    )(x, pretrained_w)


def _stage_b_body(base_ref, aT_ref, dT_ref, thr_ref, u_ref, o_ref):
    zero = jnp.zeros((), jnp.bfloat16)
    maskedT = jnp.where(aT_ref[...] >= thr_ref[...], dT_ref[...], zero)
    expert = jax.lax.dot_general(
        maskedT, u_ref[...], (((0,), (1,)), ((), ())),
        preferred_element_type=jnp.float32,
    )
    o_ref[...] = base_ref[...].astype(jnp.float32) + expert


def _stage_b(base, aT, dT, thr, u):
    grid = (_TOKENS // _TILE,)
    return pl.pallas_call(
        _stage_b_body,
        grid=grid,
        in_specs=[
            pl.BlockSpec((_TILE, _OUT), lambda i: (i, 0)),
            pl.BlockSpec((_NC, _TILE), lambda i: (0, i)),
            pl.BlockSpec((_NC, _TILE), lambda i: (0, i)),
            pl.BlockSpec((1, _TILE), lambda i: (0, i)),
            pl.BlockSpec((_OUT, _NC), lambda i: (0, 0)),
        ],
        out_specs=pl.BlockSpec((_TILE, _OUT), lambda i: (i, 0)),
        out_shape=jax.ShapeDtypeStruct((_TOKENS, _OUT), jnp.float32),
        compiler_params=pltpu.CompilerParams(
            dimension_semantics=("arbitrary",),
            vmem_limit_bytes=100 * 1024 * 1024,
        ),
    )(base, aT, dT, thr, u)


@jax.jit
def kernel(x, router_w, u, svh, pretrained_w):
    aT, dT = _stage_a(x, router_w, svh)
    thr = _sc_threshold(aT)
    base = _stage_base(x, pretrained_w)
    return _stage_b(base, aT, dT, thr.reshape(1, _TOKENS),
                    u.astype(jnp.bfloat16))


# speedup vs baseline: 1.2761x; 1.0798x over previous
import functools

import jax
import jax.numpy as jnp
from jax import lax
from jax.experimental import pallas as pl
from jax.experimental.pallas import tpu as pltpu
from jax.experimental.pallas import tpu_sc as plsc

_IN = 2048
_OUT = 2048
_NC = 1024
_TOPK = 8
_TOKENS = 8192
_TILE = 512


def _fused_body(x_ref, rw_ref, u_ref, svh_ref, pw_ref, o_ref):
    xb = x_ref[...]

    logits = jax.lax.dot_general(
        xb, rw_ref[...], (((1,), (1,)), ((), ())),
        preferred_element_type=jnp.float32,
    )
    a = jnp.abs(logits)

    cur = a
    thr = jnp.zeros((a.shape[0], 1), jnp.float32)
    for _ in range(_TOPK):
        thr = jnp.max(cur, axis=1, keepdims=True)
        cur = jnp.where(cur >= thr, -jnp.inf, cur)

    dots = jax.lax.dot_general(
        xb, svh_ref[...], (((1,), (1,)), ((), ())),
        preferred_element_type=jnp.float32,
    )
    masked = jnp.where(a >= thr, dots, 0.0)

    base = jax.lax.dot_general(
        xb, pw_ref[...], (((1,), (1,)), ((), ())),
        preferred_element_type=jnp.float32,
    )
    expert = jax.lax.dot_general(
        masked, u_ref[...], (((1,), (1,)), ((), ())),
        preferred_element_type=jnp.float32,
    )
    o_ref[...] = base + expert


def _fused_tc(x, router_w, u, svh, pretrained_w):
    grid = (_TOKENS // _TILE,)
    return pl.pallas_call(
        _fused_body,
        grid=grid,
        in_specs=[
            pl.BlockSpec((_TILE, _IN), lambda i: (i, 0)),
            pl.BlockSpec((_NC, _IN), lambda i: (0, 0)),
            pl.BlockSpec((_OUT, _NC), lambda i: (0, 0)),
            pl.BlockSpec((_NC, _IN), lambda i: (0, 0)),
            pl.BlockSpec((_OUT, _IN), lambda i: (0, 0)),
        ],
        out_specs=pl.BlockSpec((_TILE, _OUT), lambda i: (i, 0)),
        out_shape=jax.ShapeDtypeStruct((_TOKENS, _OUT), jnp.float32),
        compiler_params=pltpu.CompilerParams(
            dimension_semantics=("arbitrary",),
            vmem_limit_bytes=100 * 1024 * 1024,
        ),
    )(x, router_w, u, svh, pretrained_w)


def _stage_a_body(x_ref, rw_ref, svh_ref, aT_ref, dT_ref):
    xb = x_ref[...]
    logitsT = jax.lax.dot_general(
        rw_ref[...], xb, (((1,), (1,)), ((), ())),
        preferred_element_type=jnp.float32,
    )
    aT_ref[...] = jnp.abs(logitsT)
    dT_ref[...] = jax.lax.dot_general(
        svh_ref[...], xb, (((1,), (1,)), ((), ())),
        preferred_element_type=jnp.float32,
    ).astype(jnp.bfloat16)


def _stage_a(x, router_w, svh):
    grid = (_TOKENS // _TILE,)
    return pl.pallas_call(
        _stage_a_body,
        grid=grid,
        in_specs=[
            pl.BlockSpec((_TILE, _IN), lambda i: (i, 0)),
            pl.BlockSpec((_NC, _IN), lambda i: (0, 0)),
            pl.BlockSpec((_NC, _IN), lambda i: (0, 0)),
        ],
        out_specs=[
            pl.BlockSpec((_NC, _TILE), lambda i: (0, i)),
            pl.BlockSpec((_NC, _TILE), lambda i: (0, i)),
        ],
        out_shape=[
            jax.ShapeDtypeStruct((_NC, _TOKENS), jnp.float32),
            jax.ShapeDtypeStruct((_NC, _TOKENS), jnp.bfloat16),
        ],
        compiler_params=pltpu.CompilerParams(
            dimension_semantics=("arbitrary",),
            vmem_limit_bytes=100 * 1024 * 1024,
        ),
    )(x, router_w, svh)


_SC_LANES = 16
_SC_WORKERS = 32
_ROWS_PER_W = _TOKENS // _SC_WORKERS
_CHUNK = 128
_N_CG = _ROWS_PER_W // _CHUNK
_N_RC = _NC // _CHUNK
_N_LS = _CHUNK // _SC_LANES


def _sc_thr_kernel(aT_hbm, thr_hbm, buf0_v, buf1_v, state_v, thr_v,
                   sem0, sem1):
    wid = lax.axis_index("s") * 2 + lax.axis_index("c")
    neg_inf = jnp.full((_SC_LANES,), -jnp.inf, dtype=jnp.float32)
    bufs = (buf0_v, buf1_v)
    sems = (sem0, sem1)

    def start(cg, rc):
        col0 = pl.multiple_of((wid * _N_CG + cg) * _CHUNK, _CHUNK)
        return pltpu.async_copy(
            aT_hbm.at[pl.ds(rc * _CHUNK, _CHUNK), pl.ds(col0, _CHUNK)],
            bufs[rc % 2], sems[rc % 2],
        )

    for cg in range(_N_CG):
        for ls in range(_N_LS):
            for i in range(_TOPK):
                state_v[ls, i, :] = neg_inf
        pending = start(cg, 0)
        for rc in range(_N_RC):
            pending.wait()
            if rc + 1 < _N_RC:
                pending = start(cg, rc + 1)
            buf_v = bufs[rc % 2]
            for ls in range(_N_LS):
                def body(j, t, _b=buf_v, _ls=ls):
                    v = _b[j, pl.ds(_ls * _SC_LANES, _SC_LANES)]
                    new_t = []
                    for i in range(_TOPK):
                        hi = jnp.maximum(t[i], v)
                        v = jnp.minimum(t[i], v)
                        new_t.append(hi)
                    return tuple(new_t)

                t0 = tuple(state_v[ls, i, :] for i in range(_TOPK))
                t = lax.fori_loop(0, _CHUNK, body, t0)
                for i in range(_TOPK):
                    state_v[ls, i, :] = t[i]
        for ls in range(_N_LS):
            thr_v[pl.ds((cg * _N_LS + ls) * _SC_LANES, _SC_LANES)] = (
                state_v[ls, _TOPK - 1, :]
            )

    pltpu.sync_copy(thr_v, thr_hbm.at[pl.ds(wid * _ROWS_PER_W, _ROWS_PER_W)])


def _sc_threshold(aT):
    mesh = plsc.VectorSubcoreMesh(core_axis_name="c", subcore_axis_name="s")
    fn = functools.partial(
        pl.kernel,
        out_type=jax.ShapeDtypeStruct((_TOKENS,), jnp.float32),
        mesh=mesh,
        scratch_types=[
            pltpu.VMEM((_CHUNK, _CHUNK), jnp.float32),
            pltpu.VMEM((_CHUNK, _CHUNK), jnp.float32),
            pltpu.VMEM((_N_LS, _TOPK, _SC_LANES), jnp.float32),
            pltpu.VMEM((_ROWS_PER_W,), jnp.float32),
            pltpu.SemaphoreType.DMA,
            pltpu.SemaphoreType.DMA,
        ],
    )(_sc_thr_kernel)
    return fn(aT)


def _base_body(x_ref, pw_ref, b_ref):
    b_ref[...] = jax.lax.dot_general(
        x_ref[...], pw_ref[...], (((1,), (1,)), ((), ())),
        preferred_element_type=jnp.float32,
    ).astype(jnp.bfloat16)


def _stage_base(x, pretrained_w):
    grid = (_TOKENS // _TILE,)
    return pl.pallas_call(
        _base_body,
        grid=grid,
        in_specs=[
            pl.BlockSpec((_TILE, _IN), lambda i: (i, 0)),
            pl.BlockSpec((_OUT, _IN), lambda i: (0, 0)),
        ],
        out_specs=pl.BlockSpec((_TILE, _OUT), lambda i: (i, 0)),
        out_shape=jax.ShapeDtypeStruct((_TOKENS, _OUT), jnp.bfloat16),
        compiler_params=pltpu.CompilerParams(
            dimension_semantics=("arbitrary",),
            vmem_limit_bytes=100 * 1024 * 1024,
        ),
    )(x, pretrained_w)


def _stage_b_body(base_ref, aT_ref, dT_ref, thr_ref, u_ref, o_ref):
    zero = jnp.zeros((), jnp.bfloat16)
    maskedT = jnp.where(aT_ref[...] >= thr_ref[...], dT_ref[...], zero)
    expert = jax.lax.dot_general(
        maskedT, u_ref[...], (((0,), (1,)), ((), ())),
        preferred_element_type=jnp.float32,
    )
    o_ref[...] = base_ref[...].astype(jnp.float32) + expert


def _stage_b(base, aT, dT, thr, u):
    grid = (_TOKENS // _TILE,)
    return pl.pallas_call(
        _stage_b_body,
        grid=grid,
        in_specs=[
            pl.BlockSpec((_TILE, _OUT), lambda i: (i, 0)),
            pl.BlockSpec((_NC, _TILE), lambda i: (0, i)),
            pl.BlockSpec((_NC, _TILE), lambda i: (0, i)),
            pl.BlockSpec((1, _TILE), lambda i: (0, i)),
            pl.BlockSpec((_OUT, _NC), lambda i: (0, 0)),
        ],
        out_specs=pl.BlockSpec((_TILE, _OUT), lambda i: (i, 0)),
        out_shape=jax.ShapeDtypeStruct((_TOKENS, _OUT), jnp.float32),
        compiler_params=pltpu.CompilerParams(
            dimension_semantics=("arbitrary",),
            vmem_limit_bytes=100 * 1024 * 1024,
        ),
    )(base, aT, dT, thr, u)


@jax.jit
def kernel(x, router_w, u, svh, pretrained_w):
    aT, dT = _stage_a(x, router_w, svh)
    thr = _sc_threshold(aT)
    base = _stage_base(x, pretrained_w)
    return _stage_b(base, aT, dT, thr.reshape(1, _TOKENS),
                    u.astype(jnp.bfloat16))


# speedup vs baseline: 1.2942x; 1.0142x over previous
import functools

import jax
import jax.numpy as jnp
from jax import lax
from jax.experimental import pallas as pl
from jax.experimental.pallas import tpu as pltpu
from jax.experimental.pallas import tpu_sc as plsc

_IN = 2048
_OUT = 2048
_NC = 1024
_TOPK = 8
_TOKENS = 8192
_TILE = 1024


def _fused_body(x_ref, rw_ref, u_ref, svh_ref, pw_ref, o_ref):
    xb = x_ref[...]

    logits = jax.lax.dot_general(
        xb, rw_ref[...], (((1,), (1,)), ((), ())),
        preferred_element_type=jnp.float32,
    )
    a = jnp.abs(logits)

    cur = a
    thr = jnp.zeros((a.shape[0], 1), jnp.float32)
    for _ in range(_TOPK):
        thr = jnp.max(cur, axis=1, keepdims=True)
        cur = jnp.where(cur >= thr, -jnp.inf, cur)

    dots = jax.lax.dot_general(
        xb, svh_ref[...], (((1,), (1,)), ((), ())),
        preferred_element_type=jnp.float32,
    )
    masked = jnp.where(a >= thr, dots, 0.0)

    base = jax.lax.dot_general(
        xb, pw_ref[...], (((1,), (1,)), ((), ())),
        preferred_element_type=jnp.float32,
    )
    expert = jax.lax.dot_general(
        masked, u_ref[...], (((1,), (1,)), ((), ())),
        preferred_element_type=jnp.float32,
    )
    o_ref[...] = base + expert


def _fused_tc(x, router_w, u, svh, pretrained_w):
    grid = (_TOKENS // _TILE,)
    return pl.pallas_call(
        _fused_body,
        grid=grid,
        in_specs=[
            pl.BlockSpec((_TILE, _IN), lambda i: (i, 0)),
            pl.BlockSpec((_NC, _IN), lambda i: (0, 0)),
            pl.BlockSpec((_OUT, _NC), lambda i: (0, 0)),
            pl.BlockSpec((_NC, _IN), lambda i: (0, 0)),
            pl.BlockSpec((_OUT, _IN), lambda i: (0, 0)),
        ],
        out_specs=pl.BlockSpec((_TILE, _OUT), lambda i: (i, 0)),
        out_shape=jax.ShapeDtypeStruct((_TOKENS, _OUT), jnp.float32),
        compiler_params=pltpu.CompilerParams(
            dimension_semantics=("arbitrary",),
            vmem_limit_bytes=100 * 1024 * 1024,
        ),
    )(x, router_w, u, svh, pretrained_w)


def _stage_a_body(x_ref, rw_ref, svh_ref, aT_ref, dT_ref):
    xb = x_ref[...]
    logitsT = jax.lax.dot_general(
        rw_ref[...], xb, (((1,), (1,)), ((), ())),
        preferred_element_type=jnp.float32,
    )
    aT_ref[...] = jnp.abs(logitsT)
    dT_ref[...] = jax.lax.dot_general(
        svh_ref[...], xb, (((1,), (1,)), ((), ())),
        preferred_element_type=jnp.float32,
    ).astype(jnp.bfloat16)


def _stage_a(x, router_w, svh):
    grid = (_TOKENS // _TILE,)
    return pl.pallas_call(
        _stage_a_body,
        grid=grid,
        in_specs=[
            pl.BlockSpec((_TILE, _IN), lambda i: (i, 0)),
            pl.BlockSpec((_NC, _IN), lambda i: (0, 0)),
            pl.BlockSpec((_NC, _IN), lambda i: (0, 0)),
        ],
        out_specs=[
            pl.BlockSpec((_NC, _TILE), lambda i: (0, i)),
            pl.BlockSpec((_NC, _TILE), lambda i: (0, i)),
        ],
        out_shape=[
            jax.ShapeDtypeStruct((_NC, _TOKENS), jnp.float32),
            jax.ShapeDtypeStruct((_NC, _TOKENS), jnp.bfloat16),
        ],
        compiler_params=pltpu.CompilerParams(
            dimension_semantics=("arbitrary",),
            vmem_limit_bytes=100 * 1024 * 1024,
        ),
    )(x, router_w, svh)


_SC_LANES = 16
_SC_WORKERS = 32
_ROWS_PER_W = _TOKENS // _SC_WORKERS
_CHUNK = 128
_N_CG = _ROWS_PER_W // _CHUNK
_N_RC = _NC // _CHUNK
_N_LS = _CHUNK // _SC_LANES


def _sc_thr_kernel(aT_hbm, thr_hbm, buf0_v, buf1_v, state_v, thr_v,
                   sem0, sem1):
    wid = lax.axis_index("s") * 2 + lax.axis_index("c")
    neg_inf = jnp.full((_SC_LANES,), -jnp.inf, dtype=jnp.float32)
    bufs = (buf0_v, buf1_v)
    sems = (sem0, sem1)

    def start(cg, rc):
        col0 = pl.multiple_of((wid * _N_CG + cg) * _CHUNK, _CHUNK)
        return pltpu.async_copy(
            aT_hbm.at[pl.ds(rc * _CHUNK, _CHUNK), pl.ds(col0, _CHUNK)],
            bufs[rc % 2], sems[rc % 2],
        )

    for cg in range(_N_CG):
        for ls in range(_N_LS):
            for i in range(_TOPK):
                state_v[ls, i, :] = neg_inf
        pending = start(cg, 0)
        for rc in range(_N_RC):
            pending.wait()
            if rc + 1 < _N_RC:
                pending = start(cg, rc + 1)
            buf_v = bufs[rc % 2]
            for ls in range(_N_LS):
                def body(j, t, _b=buf_v, _ls=ls):
                    v = _b[j, pl.ds(_ls * _SC_LANES, _SC_LANES)]
                    new_t = []
                    for i in range(_TOPK):
                        hi = jnp.maximum(t[i], v)
                        v = jnp.minimum(t[i], v)
                        new_t.append(hi)
                    return tuple(new_t)

                t0 = tuple(state_v[ls, i, :] for i in range(_TOPK))
                t = lax.fori_loop(0, _CHUNK, body, t0)
                for i in range(_TOPK):
                    state_v[ls, i, :] = t[i]
        for ls in range(_N_LS):
            thr_v[pl.ds((cg * _N_LS + ls) * _SC_LANES, _SC_LANES)] = (
                state_v[ls, _TOPK - 1, :]
            )

    pltpu.sync_copy(thr_v, thr_hbm.at[pl.ds(wid * _ROWS_PER_W, _ROWS_PER_W)])


def _sc_threshold(aT):
    mesh = plsc.VectorSubcoreMesh(core_axis_name="c", subcore_axis_name="s")
    fn = functools.partial(
        pl.kernel,
        out_type=jax.ShapeDtypeStruct((_TOKENS,), jnp.float32),
        mesh=mesh,
        scratch_types=[
            pltpu.VMEM((_CHUNK, _CHUNK), jnp.float32),
            pltpu.VMEM((_CHUNK, _CHUNK), jnp.float32),
            pltpu.VMEM((_N_LS, _TOPK, _SC_LANES), jnp.float32),
            pltpu.VMEM((_ROWS_PER_W,), jnp.float32),
            pltpu.SemaphoreType.DMA,
            pltpu.SemaphoreType.DMA,
        ],
    )(_sc_thr_kernel)
    return fn(aT)


def _base_body(x_ref, pw_ref, b_ref):
    b_ref[...] = jax.lax.dot_general(
        x_ref[...], pw_ref[...], (((1,), (1,)), ((), ())),
        preferred_element_type=jnp.float32,
    ).astype(jnp.bfloat16)


def _stage_base(x, pretrained_w):
    grid = (_TOKENS // _TILE,)
    return pl.pallas_call(
        _base_body,
        grid=grid,
        in_specs=[
            pl.BlockSpec((_TILE, _IN), lambda i: (i, 0)),
            pl.BlockSpec((_OUT, _IN), lambda i: (0, 0)),
        ],
        out_specs=pl.BlockSpec((_TILE, _OUT), lambda i: (i, 0)),
        out_shape=jax.ShapeDtypeStruct((_TOKENS, _OUT), jnp.bfloat16),
        compiler_params=pltpu.CompilerParams(
            dimension_semantics=("arbitrary",),
            vmem_limit_bytes=100 * 1024 * 1024,
        ),
    )(x, pretrained_w)


def _stage_b_body(base_ref, aT_ref, dT_ref, thr_ref, u_ref, o_ref):
    zero = jnp.zeros((), jnp.bfloat16)
    maskedT = jnp.where(aT_ref[...] >= thr_ref[...], dT_ref[...], zero)
    expert = jax.lax.dot_general(
        maskedT, u_ref[...], (((0,), (1,)), ((), ())),
        preferred_element_type=jnp.float32,
    )
    o_ref[...] = base_ref[...].astype(jnp.float32) + expert


def _stage_b(base, aT, dT, thr, u):
    grid = (_TOKENS // _TILE,)
    return pl.pallas_call(
        _stage_b_body,
        grid=grid,
        in_specs=[
            pl.BlockSpec((_TILE, _OUT), lambda i: (i, 0)),
            pl.BlockSpec((_NC, _TILE), lambda i: (0, i)),
            pl.BlockSpec((_NC, _TILE), lambda i: (0, i)),
            pl.BlockSpec((1, _TILE), lambda i: (0, i)),
            pl.BlockSpec((_OUT, _NC), lambda i: (0, 0)),
        ],
        out_specs=pl.BlockSpec((_TILE, _OUT), lambda i: (i, 0)),
        out_shape=jax.ShapeDtypeStruct((_TOKENS, _OUT), jnp.float32),
        compiler_params=pltpu.CompilerParams(
            dimension_semantics=("arbitrary",),
            vmem_limit_bytes=100 * 1024 * 1024,
        ),
    )(base, aT, dT, thr, u)


@jax.jit
def kernel(x, router_w, u, svh, pretrained_w):
    aT, dT = _stage_a(x, router_w, svh)
    thr = _sc_threshold(aT)
    base = _stage_base(x, pretrained_w)
    return _stage_b(base, aT, dT, thr.reshape(1, _TOKENS),
                    u.astype(jnp.bfloat16))
